# baseline (speedup 1.0000x reference)
import functools

import numpy as np
import jax
import jax.numpy as jnp
from jax import lax
from jax.experimental import pallas as pl
from jax.experimental.pallas import tpu as pltpu
from jax.experimental.pallas import tpu_sc as plsc

_N_NODES = 100000
_N_EDGES = 3200000
_D_EDGE = 16
_K = 16
_R_MIN = 0.0
_R_MAX = 4.0

_NC = 2
_NS = 16
_NW = _NC * _NS
_CHUNK = _N_EDGES // _NW
_BLK = 800
_NBLK = _CHUNK // _BLK
_NGRP = _BLK // 16
_D_OUT = _D_EDGE + _K

_DELTA = (_R_MAX - _R_MIN) / _K
_GAMMA = np.float32(1.0 / (2.0 * _DELTA ** 2 + 1e-09))

_GATHER_DN = lax.GatherDimensionNumbers(
    offset_dims=(), collapsed_slice_dims=(0,), start_index_map=(0,))


def _lane_broadcast(v, e):
    idx = jnp.full((16, 1), e, dtype=jnp.int32)
    return lax.gather(v, idx, _GATHER_DN, (1,),
                      mode=lax.GatherScatterMode.PROMISE_IN_BOUNDS)


def _rsqrt(x):
    bi = lax.bitcast_convert_type(x, jnp.int32)
    yi = jnp.int32(0x5F3759DF) - lax.shift_right_logical(bi, 1)
    y = lax.bitcast_convert_type(yi, jnp.float32)
    h = x * jnp.float32(0.5)
    for _ in range(3):
        y = y * (jnp.float32(1.5) - h * y * y)
    return y


def _body(pos_hbm, ei_hbm, attr_hbm, out_hbm,
          row_v, col_v, ex_v, ey_v, ez_v, fx_v, fy_v, fz_v,
          rx_v, ry_v, rz_v, cx_v, cy_v, cz_v, out_v,
          pf_sh, isem, gsem, asem, osem):
    c = lax.axis_index("c")
    s = lax.axis_index("s")
    wid = s * _NC + c
    chunk0 = wid * _CHUNK

    @pl.when(s == 0)
    def _stage():
        pltpu.sync_copy(pos_hbm, pf_sh)

    plsc.subcore_barrier()

    neg_gamma = jnp.float32(-_GAMMA)
    lane = lax.iota(jnp.int32, 16)
    centers = lane.astype(jnp.float32) * jnp.float32(
        (_R_MAX - _R_MIN) / (_K - 1))

    coord = [(rx_v[0], ry_v[0], rz_v[0], cx_v[0], cy_v[0], cz_v[0]),
             (rx_v[1], ry_v[1], rz_v[1], cx_v[1], cy_v[1], cz_v[1])]
    gidx = [(ex_v[0], ey_v[0], ez_v[0], fx_v[0], fy_v[0], fz_v[0]),
            (ex_v[1], ey_v[1], ez_v[1], fx_v[1], fy_v[1], fz_v[1])]

    def fire_idx(b, p):
        base = chunk0 + b * _BLK
        pltpu.async_copy(ei_hbm.at[pl.ds(base, _BLK)], row_v[p], isem[p])
        pltpu.async_copy(ei_hbm.at[pl.ds(_N_EDGES + base, _BLK)], col_v[p],
                         isem[p])

    def drain_idx(p):
        pltpu.make_async_copy(
            ei_hbm.at[pl.ds(0, _BLK)], row_v[p], isem[p]).wait()
        pltpu.make_async_copy(
            ei_hbm.at[pl.ds(0, _BLK)], col_v[p], isem[p]).wait()

    def transform_idx(p):
        ex, ey, ez, fx, fy, fz = gidx[p]

        def tf_body(g, carry2):
            sl = pl.ds(g * 16, 16)
            r3 = row_v[p][sl] * 3
            c3 = col_v[p][sl] * 3
            ex[sl] = r3
            ey[sl] = r3 + 1
            ez[sl] = r3 + 2
            fx[sl] = c3
            fy[sl] = c3 + 1
            fz[sl] = c3 + 2
            return carry2

        lax.fori_loop(0, _NGRP, tf_body, 0)

    def fire_gathers(p):
        for t in range(6):
            pltpu.async_copy(pf_sh.at[gidx[p][t]], coord[p][t], gsem[p])

    def drain_gathers(p):
        for t in range(6):
            pltpu.make_async_copy(
                pf_sh.at[gidx[p][t]], coord[p][t], gsem[p]).wait()

    def fire_attr(b, p):
        base = chunk0 + b * _BLK
        pltpu.async_copy(attr_hbm.at[pl.ds(base, _BLK), :],
                         out_v[p].at[:, pl.ds(0, _D_EDGE)], asem[p])

    def drain_attr(p):
        pltpu.make_async_copy(attr_hbm.at[pl.ds(0, _BLK), :],
                              out_v[p].at[:, pl.ds(0, _D_EDGE)],
                              asem[p]).wait()

    def fire_out(b, p):
        base = chunk0 + b * _BLK
        pltpu.async_copy(out_v[p], out_hbm.at[pl.ds(base, _BLK), :], osem[p])

    def drain_out(p):
        pltpu.make_async_copy(out_v[p], out_hbm.at[pl.ds(0, _BLK), :],
                              osem[p]).wait()

    def compute(p):
        rx, ry, rz, cx, cy, cz = coord[p]
        ov = out_v[p]

        def grp_body(g, carry2):
            sl = pl.ds(g * 16, 16)
            dx = rx[sl] - cx[sl]
            dy = ry[sl] - cy[sl]
            dz = rz[sl] - cz[sl]
            d2 = dx * dx + dy * dy + dz * dz
            dist = d2 * _rsqrt(d2)
            for e in range(16):
                db = _lane_broadcast(dist, e)
                t = db - centers
                ov[g * 16 + e, pl.ds(_D_EDGE, _K)] = jnp.exp(
                    t * t * neg_gamma)
            return carry2

        lax.fori_loop(0, _NGRP, grp_body, 0)

    def half_step(b, p, with_out_drain):
        q = 1 - p
        nxt = jnp.minimum(b + 1, _NBLK - 1)
        nxt2 = jnp.minimum(b + 2, _NBLK - 1)
        drain_gathers(p)
        drain_idx(q)
        transform_idx(q)
        fire_gathers(q)
        fire_idx(nxt2, p)
        if with_out_drain:
            drain_out(p)
        fire_attr(b, p)
        compute(p)
        drain_attr(p)
        fire_out(b, p)
        del nxt

    pltpu.sync_copy(ei_hbm.at[pl.ds(chunk0, _BLK)], row_v[0])
    pltpu.sync_copy(ei_hbm.at[pl.ds(_N_EDGES + chunk0, _BLK)], col_v[0])
    transform_idx(0)
    fire_gathers(0)
    fire_idx(1, 1)

    half_step(jnp.int32(0), 0, False)
    half_step(jnp.int32(1), 1, False)

    def pair_body(i, carry):
        b = i * 2
        half_step(b, 0, True)
        half_step(b + 1, 1, True)
        return carry

    lax.fori_loop(1, (_NBLK - 1) // 2, pair_body, 0)

    half_step(jnp.int32(_NBLK - 1), 0, True)

    drain_gathers(1)
    drain_idx(0)
    drain_out(1)
    drain_out(0)


@jax.jit
def kernel(pos, edge_index, edge_attr):
    mesh = plsc.VectorSubcoreMesh(core_axis_name="c", subcore_axis_name="s")
    ivec = pltpu.VMEM((_BLK,), jnp.int32)
    fvec = pltpu.VMEM((_BLK,), jnp.float32)
    f = pl.kernel(
        _body,
        out_type=jax.ShapeDtypeStruct((_N_EDGES, _D_OUT), jnp.float32),
        mesh=mesh,
        scratch_types=[
            (ivec, ivec), (ivec, ivec),
            (ivec, ivec), (ivec, ivec), (ivec, ivec),
            (ivec, ivec), (ivec, ivec), (ivec, ivec),
            (fvec, fvec), (fvec, fvec), (fvec, fvec),
            (fvec, fvec), (fvec, fvec), (fvec, fvec),
            (pltpu.VMEM((_BLK, _D_OUT), jnp.float32),
             pltpu.VMEM((_BLK, _D_OUT), jnp.float32)),
            pltpu.VMEM_SHARED((3 * _N_NODES,), jnp.float32),
            (pltpu.SemaphoreType.DMA, pltpu.SemaphoreType.DMA),
            (pltpu.SemaphoreType.DMA, pltpu.SemaphoreType.DMA),
            (pltpu.SemaphoreType.DMA, pltpu.SemaphoreType.DMA),
            (pltpu.SemaphoreType.DMA, pltpu.SemaphoreType.DMA),
        ],
        compiler_params=pltpu.CompilerParams(use_tc_tiling_on_sc=False),
    )
    return f(pos.reshape(-1), edge_index.reshape(-1), edge_attr)


# speedup vs baseline: 1.0282x; 1.0282x over previous
import functools

import numpy as np
import jax
import jax.numpy as jnp
from jax import lax
from jax.experimental import pallas as pl
from jax.experimental.pallas import tpu as pltpu
from jax.experimental.pallas import tpu_sc as plsc

_N_NODES = 100000
_N_EDGES = 3200000
_D_EDGE = 16
_K = 16
_R_MIN = 0.0
_R_MAX = 4.0

_NC = 2
_NS = 16
_NW = _NC * _NS
_CHUNK = _N_EDGES // _NW
_BLK = 800
_NBLK = _CHUNK // _BLK
_NGRP = _BLK // 16
_D_OUT = _D_EDGE + _K

_DELTA = (_R_MAX - _R_MIN) / _K
_GAMMA = np.float32(1.0 / (2.0 * _DELTA ** 2 + 1e-09))

_GATHER_DN = lax.GatherDimensionNumbers(
    offset_dims=(), collapsed_slice_dims=(0,), start_index_map=(0,))


def _lane_broadcast(v, e):
    idx = jnp.full((16, 1), e, dtype=jnp.int32)
    return lax.gather(v, idx, _GATHER_DN, (1,),
                      mode=lax.GatherScatterMode.PROMISE_IN_BOUNDS)


def _rsqrt(x):
    bi = lax.bitcast_convert_type(x, jnp.int32)
    yi = jnp.int32(0x5F3759DF) - lax.shift_right_logical(bi, 1)
    y = lax.bitcast_convert_type(yi, jnp.float32)
    h = x * jnp.float32(0.5)
    for _ in range(3):
        y = y * (jnp.float32(1.5) - h * y * y)
    return y


def _body(pos_hbm, ei_hbm, attr_hbm, out_hbm,
          row_v, col_v, ex_v, ey_v, ez_v, fx_v, fy_v, fz_v,
          rx_v, ry_v, rz_v, cx_v, cy_v, cz_v, attr_v, out_v,
          pf_sh, isem, gsem, asem, osem):
    c = lax.axis_index("c")
    s = lax.axis_index("s")
    wid = s * _NC + c
    chunk0 = wid * _CHUNK

    @pl.when(s == 0)
    def _stage():
        pltpu.sync_copy(pos_hbm, pf_sh)

    plsc.subcore_barrier()

    neg_gamma = jnp.float32(-_GAMMA)
    lane = lax.iota(jnp.int32, 16)
    centers = lane.astype(jnp.float32) * jnp.float32(
        (_R_MAX - _R_MIN) / (_K - 1))

    coord = [(rx_v[0], ry_v[0], rz_v[0], cx_v[0], cy_v[0], cz_v[0]),
             (rx_v[1], ry_v[1], rz_v[1], cx_v[1], cy_v[1], cz_v[1])]
    gidx = [(ex_v[0], ey_v[0], ez_v[0], fx_v[0], fy_v[0], fz_v[0]),
            (ex_v[1], ey_v[1], ez_v[1], fx_v[1], fy_v[1], fz_v[1])]

    def fire_idx(b, p):
        base = chunk0 + b * _BLK
        pltpu.async_copy(ei_hbm.at[pl.ds(base, _BLK)], row_v[p], isem[p])
        pltpu.async_copy(ei_hbm.at[pl.ds(_N_EDGES + base, _BLK)], col_v[p],
                         isem[p])

    def drain_idx(p):
        pltpu.make_async_copy(
            ei_hbm.at[pl.ds(0, _BLK)], row_v[p], isem[p]).wait()
        pltpu.make_async_copy(
            ei_hbm.at[pl.ds(0, _BLK)], col_v[p], isem[p]).wait()

    def transform_idx(p):
        ex, ey, ez, fx, fy, fz = gidx[p]

        def tf_body(g, carry2):
            sl = pl.ds(g * 16, 16)
            r3 = row_v[p][sl] * 3
            c3 = col_v[p][sl] * 3
            ex[sl] = r3
            ey[sl] = r3 + 1
            ez[sl] = r3 + 2
            fx[sl] = c3
            fy[sl] = c3 + 1
            fz[sl] = c3 + 2
            return carry2

        lax.fori_loop(0, _NGRP, tf_body, 0)

    def fire_gathers(p):
        for t in range(6):
            pltpu.async_copy(pf_sh.at[gidx[p][t]], coord[p][t], gsem[p])

    def drain_gathers(p):
        for t in range(6):
            pltpu.make_async_copy(
                pf_sh.at[gidx[p][t]], coord[p][t], gsem[p]).wait()

    def fire_attr(b, p):
        base = chunk0 + b * _BLK
        pltpu.async_copy(attr_hbm.at[pl.ds(base * _D_EDGE, _BLK * _D_EDGE)],
                         attr_v[p], asem[p])

    def drain_attr(p):
        pltpu.make_async_copy(attr_hbm.at[pl.ds(0, _BLK * _D_EDGE)],
                              attr_v[p], asem[p]).wait()

    def fire_out(b, p):
        base = chunk0 + b * _BLK
        pltpu.async_copy(out_v[p], out_hbm.at[pl.ds(base * _D_OUT,
                                                    _BLK * _D_OUT)], osem[p])

    def drain_out(p):
        pltpu.make_async_copy(out_v[p], out_hbm.at[pl.ds(0, _BLK * _D_OUT)],
                              osem[p]).wait()

    def compute(p):
        rx, ry, rz, cx, cy, cz = coord[p]
        ov = out_v[p]
        av = attr_v[p]

        def grp_body(g, carry2):
            sl = pl.ds(g * 16, 16)
            dx = rx[sl] - cx[sl]
            dy = ry[sl] - cy[sl]
            dz = rz[sl] - cz[sl]
            d2 = dx * dx + dy * dy + dz * dz
            dist = d2 * _rsqrt(d2)
            e0 = g * 16
            for e in range(16):
                db = _lane_broadcast(dist, e)
                t = db - centers
                ov[pl.ds((e0 + e) * _D_OUT + _D_EDGE, _K)] = jnp.exp(
                    t * t * neg_gamma)
                ov[pl.ds((e0 + e) * _D_OUT, _D_EDGE)] = av[
                    pl.ds((e0 + e) * _D_EDGE, _D_EDGE)]
            return carry2

        lax.fori_loop(0, _NGRP, grp_body, 0)

    def half_step(b, p, with_out_drain):
        q = 1 - p
        nxt = jnp.minimum(b + 1, _NBLK - 1)
        nxt2 = jnp.minimum(b + 2, _NBLK - 1)
        drain_gathers(p)
        drain_idx(q)
        transform_idx(q)
        fire_gathers(q)
        fire_idx(nxt2, p)
        if with_out_drain:
            drain_out(p)
        drain_attr(p)
        compute(p)
        fire_attr(nxt2, p)
        fire_out(b, p)
        del nxt

    pltpu.sync_copy(ei_hbm.at[pl.ds(chunk0, _BLK)], row_v[0])
    pltpu.sync_copy(ei_hbm.at[pl.ds(_N_EDGES + chunk0, _BLK)], col_v[0])
    transform_idx(0)
    fire_gathers(0)
    fire_idx(1, 1)
    fire_attr(0, 0)
    fire_attr(1, 1)

    half_step(jnp.int32(0), 0, False)
    half_step(jnp.int32(1), 1, False)

    def pair_body(i, carry):
        b = i * 2
        half_step(b, 0, True)
        half_step(b + 1, 1, True)
        return carry

    lax.fori_loop(1, (_NBLK - 1) // 2, pair_body, 0)

    half_step(jnp.int32(_NBLK - 1), 0, True)

    drain_gathers(1)
    drain_idx(0)
    drain_attr(0)
    drain_attr(1)
    drain_out(1)
    drain_out(0)


@jax.jit
def kernel(pos, edge_index, edge_attr):
    mesh = plsc.VectorSubcoreMesh(core_axis_name="c", subcore_axis_name="s")
    ivec = pltpu.VMEM((_BLK,), jnp.int32)
    fvec = pltpu.VMEM((_BLK,), jnp.float32)
    f = pl.kernel(
        _body,
        out_type=jax.ShapeDtypeStruct((_N_EDGES * _D_OUT,), jnp.float32),
        mesh=mesh,
        scratch_types=[
            (ivec, ivec), (ivec, ivec),
            (ivec, ivec), (ivec, ivec), (ivec, ivec),
            (ivec, ivec), (ivec, ivec), (ivec, ivec),
            (fvec, fvec), (fvec, fvec), (fvec, fvec),
            (fvec, fvec), (fvec, fvec), (fvec, fvec),
            (pltpu.VMEM((_BLK * _D_EDGE,), jnp.float32),
             pltpu.VMEM((_BLK * _D_EDGE,), jnp.float32)),
            (pltpu.VMEM((_BLK * _D_OUT,), jnp.float32),
             pltpu.VMEM((_BLK * _D_OUT,), jnp.float32)),
            pltpu.VMEM_SHARED((3 * _N_NODES,), jnp.float32),
            (pltpu.SemaphoreType.DMA, pltpu.SemaphoreType.DMA),
            (pltpu.SemaphoreType.DMA, pltpu.SemaphoreType.DMA),
            (pltpu.SemaphoreType.DMA, pltpu.SemaphoreType.DMA),
            (pltpu.SemaphoreType.DMA, pltpu.SemaphoreType.DMA),
        ],
        compiler_params=pltpu.CompilerParams(use_tc_tiling_on_sc=False),
    )
    out = f(pos.reshape(-1), edge_index.reshape(-1), edge_attr.reshape(-1))
    return out.reshape(_N_EDGES, _D_OUT)
